# Initial kernel scaffold; baseline (speedup 1.0000x reference)
import jax
import jax.numpy as jnp
from jax.experimental import pallas as pl


def kernel(gh_label, gah_label, text_map, link_map, conf_map, a_logits, p_logits, a_label, p_label, log_probs, targets, target_lengths):
    raise NotImplementedError("write your pallas kernel here")


# speedup vs baseline: 34.4643x; 34.4643x over previous
import functools

import jax
import jax.numpy as jnp
from jax import lax
from jax.experimental import pallas as pl
from jax.experimental.pallas import tpu as pltpu

NEG = -1e9
_PN = 384 * 384
_TOPINF = 0x7F800000


def _row_sum(x):
    return jnp.sum(x, axis=(1, 2), keepdims=True)


def _topk_sum(V, M, k, lo):
    t = lax.bitcast_convert_type(lo, jnp.float32)
    gt = jnp.logical_and(M, V > t)
    sum_gt = _row_sum(jnp.where(gt, V, 0.0))
    cnt_gt = _row_sum(jnp.where(gt, 1.0, 0.0))
    return sum_gt + (k - cnt_gt) * t


def _loss_kernel(gh_ref, gah_ref, text_ref, link_ref, conf_ref,
                 a_log_ref, p_log_ref, a_lab_ref, p_lab_ref,
                 lpt_ref, ext_ref, skip_ref, tlen_ref, out_ref, lpe_ref):
    conf = conf_ref[...]
    gh = gh_ref[...]
    gah = gah_ref[...]
    loss_g = (text_ref[...] - gh) ** 2 * conf
    loss_a = (link_ref[...] - gah) ** 2 * conf

    pos_g = gh >= 0.1
    pos_a = gah >= 0.1
    neg_g = jnp.logical_not(pos_g)
    neg_a = jnp.logical_not(pos_a)

    pcnt_g = _row_sum(jnp.where(pos_g, 1.0, 0.0))
    pcnt_a = _row_sum(jnp.where(pos_a, 1.0, 0.0))
    spos_g = _row_sum(jnp.where(pos_g, loss_g, 0.0))
    spos_a = _row_sum(jnp.where(pos_a, loss_a, 0.0))
    sneg_g = _row_sum(jnp.where(neg_g, loss_g, 0.0))
    sneg_a = _row_sum(jnp.where(neg_a, loss_a, 0.0))
    ncnt_g = float(_PN) - pcnt_g
    ncnt_a = float(_PN) - pcnt_a
    k3_g = 3.0 * pcnt_g
    k3_a = 3.0 * pcnt_a
    k500 = jnp.full_like(pcnt_g, 500.0)

    B = gh.shape[0]
    zero = jnp.zeros((B, 1, 1), jnp.int32)
    top = jnp.full((B, 1, 1), _TOPINF, jnp.int32)

    def bs_step(_, state):
        lo_g, hi_g, lo_a, hi_a, lo_G, hi_G, lo_A, hi_A = state

        def upd(lo, hi, V, M, k):
            mid = lo + lax.shift_right_logical(hi - lo, 1)
            t = lax.bitcast_convert_type(mid, jnp.float32)
            cnt = _row_sum(jnp.where(jnp.logical_and(M, V >= t), 1.0, 0.0))
            ge = cnt >= k
            return jnp.where(ge, mid, lo), jnp.where(ge, hi, mid)

        lo_g, hi_g = upd(lo_g, hi_g, loss_g, neg_g, k3_g)
        lo_a, hi_a = upd(lo_a, hi_a, loss_a, neg_a, k3_a)
        lo_G, hi_G = upd(lo_G, hi_G, loss_g, True, k500)
        lo_A, hi_A = upd(lo_A, hi_A, loss_a, True, k500)
        return lo_g, hi_g, lo_a, hi_a, lo_G, hi_G, lo_A, hi_A

    state = (zero, top, zero, top, zero, top, zero, top)
    state = lax.fori_loop(0, 31, bs_step, state)
    lo_g, _, lo_a, _, lo_G, _, lo_A, _ = state

    tk_g = _topk_sum(loss_g, neg_g, k3_g, lo_g)
    tk_a = _topk_sum(loss_a, neg_a, k3_a, lo_a)
    t500_g = _topk_sum(loss_g, True, k500, lo_G)
    t500_a = _topk_sum(loss_a, True, k500, lo_A)

    def contrib(pcnt, ncnt, spos, sneg, k3, tk, t500):
        posi = spos / jnp.maximum(pcnt, 1.0)
        mean_neg = sneg / jnp.maximum(ncnt, 1.0)
        topk_neg = tk / jnp.maximum(k3, 1.0)
        nega = jnp.where(ncnt < k3, mean_neg, topk_neg)
        c = jnp.where(pcnt > 0, posi + nega, t500 / 500.0)
        return jnp.sum(c)

    char_loss = contrib(pcnt_g, ncnt_g, spos_g, sneg_g, k3_g, tk_g, t500_g)
    affi_loss = contrib(pcnt_a, ncnt_a, spos_a, sneg_a, k3_a, tk_a, t500_a)
    saliency = (char_loss + affi_loss) / float(B)

    def ce(logits, labels2d):
        n, c = logits.shape
        m = jnp.max(logits, axis=1, keepdims=True)
        ls = logits - m - jnp.log(jnp.sum(jnp.exp(logits - m), axis=1, keepdims=True))
        oh = lax.broadcasted_iota(jnp.int32, (n, c), 1) == labels2d
        return -jnp.sum(jnp.where(oh, ls, 0.0)) / float(n)

    direction = 0.5 * ce(p_log_ref[...], p_lab_ref[...]) + \
        0.5 * ce(a_log_ref[...], a_lab_ref[...])

    lpt = lpt_ref[...]
    ext = ext_ref[...]
    skipf = skip_ref[...]
    tlen = tlen_ref[...]
    N, T, C = lpt.shape
    L = ext.shape[1]

    oh = (ext[:, :, None] == lax.broadcasted_iota(jnp.int32, (N, L, C), 2))
    oh = oh.astype(jnp.float32)
    cols = []
    for n_i in range(N):
        cols.append(lax.dot_general(lpt[n_i], oh[n_i],
                                    (((1,), (1,)), ((), ())),
                                    precision=lax.Precision.HIGHEST)[:, None, :])
    lpe_ref[...] = jnp.concatenate(cols, axis=1)

    li = lax.broadcasted_iota(jnp.int32, (N, L), 1)

    alpha0 = jnp.where(li <= 1, lpe_ref[0], NEG)

    def ctc_step(t, alpha):
        lp_t = lpe_ref[pl.ds(t, 1)].reshape(N, L)
        a1 = jnp.where(li >= 1, pltpu.roll(alpha, 1, 1), NEG)
        a2 = jnp.where(li >= 2, pltpu.roll(alpha, 2, 1), NEG)
        a2 = jnp.where(skipf > 0, a2, NEG)
        m = jnp.maximum(jnp.maximum(alpha, a1), a2)
        new = m + jnp.log(jnp.exp(alpha - m) + jnp.exp(a1 - m) + jnp.exp(a2 - m))
        new = new + lp_t
        return jnp.maximum(new, NEG)

    alpha = lax.fori_loop(1, T, ctc_step, alpha0)

    tl_i = tlen.astype(jnp.int32)
    i1 = jnp.clip(2 * tl_i, 0, L - 1)
    i2 = jnp.clip(2 * tl_i - 1, 0, L - 1)
    v1 = jnp.sum(jnp.where(li == i1, alpha, 0.0), axis=1, keepdims=True)
    v2 = jnp.sum(jnp.where(li == i2, alpha, 0.0), axis=1, keepdims=True)
    m = jnp.maximum(v1, v2)
    ll = m + jnp.log(jnp.exp(v1 - m) + jnp.exp(v2 - m))
    closs = -ll
    closs = jnp.where(closs < 1e8, closs, 0.0)
    recognition = 10.0 * jnp.mean(closs / jnp.maximum(tlen, 1.0))

    total = saliency + recognition
    lane = lax.broadcasted_iota(jnp.int32, (8, 128), 1)
    out = (jnp.where(lane == 0, total, 0.0) + jnp.where(lane == 1, saliency, 0.0)
           + jnp.where(lane == 2, direction, 0.0)
           + jnp.where(lane == 3, recognition, 0.0))
    out_ref[...] = out


@functools.partial(jax.jit, static_argnames=("interpret",))
def _run(gh_label, gah_label, text_map, link_map, conf_map, a_logits, p_logits,
         a_label, p_label, log_probs, targets, target_lengths, interpret=False):
    N, S = targets.shape
    L = 2 * S + 1
    ext = jnp.zeros((N, L), dtype=targets.dtype)
    ext = ext.at[:, 1::2].set(targets)
    prev2 = jnp.concatenate(
        [jnp.full((N, 2), -1, dtype=ext.dtype), ext[:, :-2]], axis=1)
    allow_skip = ((ext != 0) & (ext != prev2)).astype(jnp.float32)
    lpt = jnp.transpose(log_probs, (1, 0, 2))
    tlen = target_lengths.astype(jnp.float32)[:, None]

    T = log_probs.shape[0]
    out = pl.pallas_call(
        _loss_kernel,
        out_shape=jax.ShapeDtypeStruct((8, 128), jnp.float32),
        scratch_shapes=[pltpu.VMEM((T, N, L), jnp.float32)],
        interpret=interpret,
    )(gh_label, gah_label, text_map, link_map, conf_map,
      a_logits, p_logits, a_label[:, None], p_label[:, None],
      lpt, ext, allow_skip, tlen)
    return out[0, 0], out[0, 1], out[0, 2], out[0, 3]


def kernel(gh_label, gah_label, text_map, link_map, conf_map, a_logits,
           p_logits, a_label, p_label, log_probs, targets, target_lengths):
    return _run(gh_label, gah_label, text_map, link_map, conf_map, a_logits,
                p_logits, a_label, p_label, log_probs, targets, target_lengths)
